# Initial kernel scaffold; baseline (speedup 1.0000x reference)
import functools

import jax
import jax.numpy as jnp
from jax import lax
from jax.experimental import pallas as pl
from jax.experimental.pallas import tpu as pltpu
from jax.experimental.pallas import tpu_sc as plsc

N = 10000
E = 320000
D = 128
DP = 144

NC = 2
NS = 16
NW = NC * NS
EPW = E // NW
CH = 80
NCH = EPW // CH
RPT = N // NS
RCH = 125
NRC = RPT // RCH


def _zero_fill(zbuf, rows, width):
    lanes = width // 16

    def body(i, _):
        r = i // lanes
        c = (i % lanes) * 16
        zbuf[r, pl.ds(c, 16)] = jnp.zeros((16,), jnp.float32)
        return 0

    lax.fori_loop(0, rows * lanes, body, 0)


def _sc_pass_a(ea_hbm, rec3d_hbm):
    mesh = plsc.VectorSubcoreMesh(core_axis_name="c", subcore_axis_name="s")

    @functools.partial(
        pl.kernel,
        mesh=mesh,
        out_type=jax.ShapeDtypeStruct((NC * N, D), jnp.float32),
        scratch_types=[
            pltpu.VMEM_SHARED((N, D), jnp.float32),
            pltpu.VMEM((RCH, D), jnp.float32),
            pltpu.VMEM((NCH, CH), jnp.int32),
            pltpu.VMEM((CH, D), jnp.float32),
            pltpu.SemaphoreType.DMA,
        ],
    )
    def k(ea, rec3d, out, acc, zbuf, ridx, rows, sem):
        cid = lax.axis_index("c")
        sid = lax.axis_index("s")
        wid = cid * NS + sid

        _zero_fill(zbuf, RCH, D)
        for j in range(NRC):
            pltpu.sync_copy(zbuf, acc.at[pl.ds(sid * RPT + j * RCH, RCH)])
        plsc.subcore_barrier()

        pltpu.sync_copy(rec3d.at[wid], ridx)

        def body(c, _):
            pltpu.async_copy(ea.at[pl.ds(wid * EPW + c * CH, CH)], rows, sem).wait()
            pltpu.sync_copy(rows, acc.at[ridx.at[c]], add=True)
            return 0

        lax.fori_loop(0, NCH, body, 0)
        plsc.subcore_barrier()

        for j in range(NRC):
            r0 = sid * RPT + j * RCH
            pltpu.sync_copy(acc.at[pl.ds(r0, RCH)], zbuf)
            pltpu.sync_copy(zbuf, out.at[pl.ds(cid * N + r0, RCH)])

    return k(ea_hbm, rec3d_hbm)


def _sc_pass_b(g1p_hbm, send3d_hbm, rec3d_hbm):
    mesh = plsc.VectorSubcoreMesh(core_axis_name="c", subcore_axis_name="s")

    @functools.partial(
        pl.kernel,
        mesh=mesh,
        out_type=jax.ShapeDtypeStruct((NC * N, DP), jnp.float32),
        scratch_types=[
            pltpu.VMEM_SHARED((N, DP), jnp.float32),
            pltpu.VMEM((RCH, DP), jnp.float32),
            pltpu.VMEM((NCH, CH), jnp.int32),
            pltpu.VMEM((NCH, CH), jnp.int32),
            pltpu.VMEM((CH, DP), jnp.float32),
            pltpu.SemaphoreType.DMA,
        ],
    )
    def k(g1p, send3d, rec3d, out, acc, zbuf, sidx, ridx, rows, sem):
        cid = lax.axis_index("c")
        sid = lax.axis_index("s")
        wid = cid * NS + sid

        _zero_fill(zbuf, RCH, DP)
        for j in range(NRC):
            pltpu.sync_copy(zbuf, acc.at[pl.ds(sid * RPT + j * RCH, RCH)])
        plsc.subcore_barrier()

        pltpu.sync_copy(send3d.at[wid], sidx)
        pltpu.sync_copy(rec3d.at[wid], ridx)

        def body(c, _):
            pltpu.async_copy(g1p.at[sidx.at[c]], rows, sem).wait()
            pltpu.sync_copy(rows, acc.at[ridx.at[c]], add=True)
            return 0

        lax.fori_loop(0, NCH, body, 0)
        plsc.subcore_barrier()

        for j in range(NRC):
            r0 = sid * RPT + j * RCH
            pltpu.sync_copy(acc.at[pl.ds(r0, RCH)], zbuf)
            pltpu.sync_copy(zbuf, out.at[pl.ds(cid * N + r0, RCH)])

    return k(g1p_hbm, send3d_hbm, rec3d_hbm)


def _tc_prep(h, W_msg, W_upd):
    def body(h_ref, wm_ref, wu_ref, g1p_ref):
        Wu_m = wu_ref[D : 2 * D, :]
        Wsp = jnp.dot(wm_ref[0:D, :], Wu_m, preferred_element_type=jnp.float32)
        g = jnp.dot(h_ref[...], Wsp, preferred_element_type=jnp.float32)
        pad = jnp.concatenate(
            [jnp.ones((N, 1), jnp.float32), jnp.zeros((N, DP - D - 1), jnp.float32)],
            axis=1,
        )
        g1p_ref[...] = jnp.concatenate([g, pad], axis=1)

    return pl.pallas_call(
        body,
        out_shape=jax.ShapeDtypeStruct((N, DP), jnp.float32),
    )(h, W_msg, W_upd)


def _tc_final(h, pA, pB, W_msg, b_msg2, W_upd, b_upd2):
    def body(h_ref, pA_ref, pB_ref, wm_ref, bm_ref, wu_ref, bu_ref, out_ref):
        Wu_h = wu_ref[0:D, :]
        Wu_m = wu_ref[D : 2 * D, :]
        Wrp = jnp.dot(wm_ref[D : 2 * D, :], Wu_m, preferred_element_type=jnp.float32)
        Wep = jnp.dot(wm_ref[2 * D : 3 * D, :], Wu_m, preferred_element_type=jnp.float32)
        bp = jnp.dot(bm_ref[...], Wu_m, preferred_element_type=jnp.float32)
        S3 = pA_ref[0:N, :] + pA_ref[N : 2 * N, :]
        SgD = pB_ref[0:N, :] + pB_ref[N : 2 * N, :]
        Sg = SgD[:, 0:D]
        deg = SgD[:, D : D + 1]
        hv = h_ref[...]
        out = (
            jnp.dot(hv, Wu_h, preferred_element_type=jnp.float32)
            + Sg
            + jnp.dot(S3, Wep, preferred_element_type=jnp.float32)
            + deg * (jnp.dot(hv, Wrp, preferred_element_type=jnp.float32) + bp)
            + bu_ref[...]
        )
        out_ref[...] = out

    return pl.pallas_call(
        body,
        out_shape=jax.ShapeDtypeStruct((N, D), jnp.float32),
    )(h, pA, pB, W_msg, b_msg2, W_upd, b_upd2)


def kernel(h, edge_index, edge_attr, W_msg, b_msg, W_upd, b_upd):
    send = edge_index[0].astype(jnp.int32)
    rec = edge_index[1].astype(jnp.int32)
    send3d = send.reshape(NW, NCH, CH)
    rec3d = rec.reshape(NW, NCH, CH)

    g1p = _tc_prep(h, W_msg, W_upd)
    pA = _sc_pass_a(edge_attr, rec3d)
    pB = _sc_pass_b(g1p, send3d, rec3d)
    return _tc_final(
        h, pA, pB, W_msg, b_msg.reshape(1, D), W_upd, b_upd.reshape(1, D)
    )


# speedup vs baseline: 4.4428x; 4.4428x over previous
import functools

import jax
import jax.numpy as jnp
from jax import lax
from jax.experimental import pallas as pl
from jax.experimental.pallas import tpu as pltpu
from jax.experimental.pallas import tpu_sc as plsc

N = 10000
E = 320000
D = 128

NC = 2
NS = 16
NW = NC * NS
EPW = E // NW
CH = 80
NCH = EPW // CH
RCH = 80
NRC = N // RCH
RITER = -(-NRC // NS)


def _zero_fill(zbuf, rows, width):
    lanes = width // 16

    def body(i, _):
        r = i // lanes
        c = (i % lanes) * 16
        zbuf[r, pl.ds(c, 16)] = jnp.zeros((16,), jnp.float32)
        return 0

    lax.fori_loop(0, rows * lanes, body, 0)


def _sc_pass_a(ea_hbm, rec3d_hbm):
    mesh = plsc.VectorSubcoreMesh(core_axis_name="c", subcore_axis_name="s")

    @functools.partial(
        pl.kernel,
        mesh=mesh,
        out_type=jax.ShapeDtypeStruct((NC * N, D), jnp.float32),
        scratch_types=[
            pltpu.VMEM_SHARED((N, D), jnp.float32),
            pltpu.VMEM((NCH, CH), jnp.int32),
            pltpu.VMEM((CH, D), jnp.float32),
            pltpu.SemaphoreType.DMA,
        ],
    )
    def k(ea, rec3d, out, acc, ridx, rows, sem):
        cid = lax.axis_index("c")
        sid = lax.axis_index("s")
        wid = cid * NS + sid

        _zero_fill(rows, RCH, D)

        def zinit(j, _):
            rc = sid + j * NS

            @pl.when(rc < NRC)
            def _():
                pltpu.sync_copy(rows, acc.at[pl.ds(rc * RCH, RCH)])

            return 0

        lax.fori_loop(0, RITER, zinit, 0)
        plsc.subcore_barrier()

        pltpu.sync_copy(rec3d.at[wid], ridx)

        def body(c, _):
            pltpu.async_copy(ea.at[pl.ds(wid * EPW + c * CH, CH)], rows, sem).wait()
            pltpu.sync_copy(rows, acc.at[ridx.at[c]], add=True)
            return 0

        lax.fori_loop(0, NCH, body, 0)
        plsc.subcore_barrier()

        def wout(j, _):
            rc = sid + j * NS

            @pl.when(rc < NRC)
            def _():
                pltpu.sync_copy(acc.at[pl.ds(rc * RCH, RCH)], rows)
                pltpu.sync_copy(rows, out.at[pl.ds(cid * N + rc * RCH, RCH)])

            return 0

        lax.fori_loop(0, RITER, wout, 0)

    return k(ea_hbm, rec3d_hbm)


def _sc_pass_b(g1_hbm, g2_hbm, send3d_hbm, rec3d_hbm):
    mesh = plsc.VectorSubcoreMesh(core_axis_name="c", subcore_axis_name="s")

    @functools.partial(
        pl.kernel,
        mesh=mesh,
        out_type=jax.ShapeDtypeStruct((NC * N, D), jnp.float32),
        scratch_types=[
            pltpu.VMEM_SHARED((N, D), jnp.float32),
            pltpu.VMEM((NCH, CH), jnp.int32),
            pltpu.VMEM((NCH, CH), jnp.int32),
            pltpu.VMEM((CH, D), jnp.float32),
            pltpu.SemaphoreType.DMA,
        ],
    )
    def k(g1, g2, send3d, rec3d, out, acc, sidx, ridx, rows, sem):
        cid = lax.axis_index("c")
        sid = lax.axis_index("s")
        wid = cid * NS + sid

        _zero_fill(rows, CH, D)

        def zinit(j, _):
            rc = sid + j * NS

            @pl.when(rc < NRC)
            def _():
                pltpu.sync_copy(rows, acc.at[pl.ds(rc * RCH, RCH)])

            return 0

        lax.fori_loop(0, RITER, zinit, 0)
        plsc.subcore_barrier()

        pltpu.sync_copy(send3d.at[wid], sidx)
        pltpu.sync_copy(rec3d.at[wid], ridx)

        def body1(c, _):
            pltpu.async_copy(g1.at[sidx.at[c]], rows, sem).wait()
            pltpu.sync_copy(rows, acc.at[ridx.at[c]], add=True)
            return 0

        lax.fori_loop(0, NCH, body1, 0)

        def body2(c, _):
            pltpu.async_copy(g2.at[ridx.at[c]], rows, sem).wait()
            pltpu.sync_copy(rows, acc.at[ridx.at[c]], add=True)
            return 0

        lax.fori_loop(0, NCH, body2, 0)
        plsc.subcore_barrier()

        def wout(j, _):
            rc = sid + j * NS

            @pl.when(rc < NRC)
            def _():
                pltpu.sync_copy(acc.at[pl.ds(rc * RCH, RCH)], rows)
                pltpu.sync_copy(rows, out.at[pl.ds(cid * N + rc * RCH, RCH)])

            return 0

        lax.fori_loop(0, RITER, wout, 0)

    return k(g1_hbm, g2_hbm, send3d_hbm, rec3d_hbm)


def _tc_prep(h, W_msg, b_msg2, W_upd):
    def body(h_ref, wm_ref, bm_ref, wu_ref, g1_ref, g2_ref):
        Wu_m = wu_ref[D : 2 * D, :]
        Wsp = jnp.dot(wm_ref[0:D, :], Wu_m, preferred_element_type=jnp.float32)
        Wrp = jnp.dot(wm_ref[D : 2 * D, :], Wu_m, preferred_element_type=jnp.float32)
        bp = jnp.dot(bm_ref[...], Wu_m, preferred_element_type=jnp.float32)
        hv = h_ref[...]
        g1_ref[...] = jnp.dot(hv, Wsp, preferred_element_type=jnp.float32)
        g2_ref[...] = jnp.dot(hv, Wrp, preferred_element_type=jnp.float32) + bp

    return pl.pallas_call(
        body,
        out_shape=(
            jax.ShapeDtypeStruct((N, D), jnp.float32),
            jax.ShapeDtypeStruct((N, D), jnp.float32),
        ),
    )(h, W_msg, b_msg2, W_upd)


def _tc_final(h, pA, pB, W_msg, W_upd, b_upd2):
    def body(h_ref, pA_ref, pB_ref, wm_ref, wu_ref, bu_ref, out_ref):
        Wu_h = wu_ref[0:D, :]
        Wu_m = wu_ref[D : 2 * D, :]
        Wep = jnp.dot(wm_ref[2 * D : 3 * D, :], Wu_m, preferred_element_type=jnp.float32)
        S3 = pA_ref[0:N, :] + pA_ref[N : 2 * N, :]
        Sg12 = pB_ref[0:N, :] + pB_ref[N : 2 * N, :]
        out = (
            jnp.dot(h_ref[...], Wu_h, preferred_element_type=jnp.float32)
            + Sg12
            + jnp.dot(S3, Wep, preferred_element_type=jnp.float32)
            + bu_ref[...]
        )
        out_ref[...] = out

    return pl.pallas_call(
        body,
        out_shape=jax.ShapeDtypeStruct((N, D), jnp.float32),
    )(h, pA, pB, W_msg, W_upd, b_upd2)


def kernel(h, edge_index, edge_attr, W_msg, b_msg, W_upd, b_upd):
    send = edge_index[0].astype(jnp.int32)
    rec = edge_index[1].astype(jnp.int32)
    send3d = send.reshape(NW, NCH, CH)
    rec3d = rec.reshape(NW, NCH, CH)

    g1, g2 = _tc_prep(h, W_msg, b_msg.reshape(1, D), W_upd)
    pA = _sc_pass_a(edge_attr, rec3d)
    pB = _sc_pass_b(g1, g2, send3d, rec3d)
    return _tc_final(h, pA, pB, W_msg, W_upd, b_upd.reshape(1, D))


# speedup vs baseline: 7.2195x; 1.6250x over previous
import functools

import jax
import jax.numpy as jnp
from jax import lax
from jax.experimental import pallas as pl
from jax.experimental.pallas import tpu as pltpu
from jax.experimental.pallas import tpu_sc as plsc

N = 10000
E = 320000
D = 128

NC = 2
NS = 16
NW = NC * NS
EPW = E // NW
CH = 80
NCH = EPW // CH
RCH = 80
NRC = N // RCH
RITER = -(-NRC // NS)


def _zero_fill(zbuf, rows, width):
    lanes = width // 16

    def body(i, _):
        r = i // lanes
        c = (i % lanes) * 16
        zbuf[r, pl.ds(c, 16)] = jnp.zeros((16,), jnp.float32)
        return 0

    lax.fori_loop(0, rows * lanes, body, 0)


def _sc_pass_a(ea_hbm, rec3d_hbm):
    mesh = plsc.VectorSubcoreMesh(core_axis_name="c", subcore_axis_name="s")

    @functools.partial(
        pl.kernel,
        mesh=mesh,
        out_type=jax.ShapeDtypeStruct((NC * N, D), jnp.float32),
        scratch_types=[
            pltpu.VMEM_SHARED((N, D), jnp.float32),
            pltpu.VMEM((NCH, CH), jnp.int32),
            pltpu.VMEM((CH, D), jnp.float32),
            pltpu.VMEM((CH, D), jnp.float32),
            pltpu.SemaphoreType.DMA,
            pltpu.SemaphoreType.DMA,
        ],
    )
    def k(ea, rec3d, out, acc, ridx, rowsA, rowsB, semA, semB):
        cid = lax.axis_index("c")
        sid = lax.axis_index("s")
        wid = cid * NS + sid
        base = wid * EPW

        _zero_fill(rowsA, CH, D)

        def zinit(j, _):
            rc = sid + j * NS

            @pl.when(rc < NRC)
            def _():
                pltpu.sync_copy(rowsA, acc.at[pl.ds(rc * RCH, RCH)])

            return 0

        lax.fori_loop(0, RITER, zinit, 0)
        plsc.subcore_barrier()

        pltpu.sync_copy(rec3d.at[wid], ridx)

        pltpu.async_copy(ea.at[pl.ds(base, CH)], rowsA, semA)

        def pair(j, _):
            c0 = 2 * j
            c1 = c0 + 1
            pltpu.async_copy(ea.at[pl.ds(base + c1 * CH, CH)], rowsB, semB)
            pltpu.make_async_copy(ea.at[pl.ds(base, CH)], rowsA, semA).wait()
            pltpu.sync_copy(rowsA, acc.at[ridx.at[c0]], add=True)
            pltpu.async_copy(ea.at[pl.ds(base + (c0 + 2) * CH, CH)], rowsA, semA)
            pltpu.make_async_copy(ea.at[pl.ds(base, CH)], rowsB, semB).wait()
            pltpu.sync_copy(rowsB, acc.at[ridx.at[c1]], add=True)
            return 0

        lax.fori_loop(0, NCH // 2, pair, 0)
        pltpu.make_async_copy(ea.at[pl.ds(base, CH)], rowsA, semA).wait()
        pltpu.sync_copy(rowsA, acc.at[ridx.at[NCH - 1]], add=True)
        plsc.subcore_barrier()

        def wout(j, _):
            rc = sid + j * NS

            @pl.when(rc < NRC)
            def _():
                pltpu.sync_copy(acc.at[pl.ds(rc * RCH, RCH)], rowsA)
                pltpu.sync_copy(rowsA, out.at[pl.ds(cid * N + rc * RCH, RCH)])

            return 0

        lax.fori_loop(0, RITER, wout, 0)

    return k(ea_hbm, rec3d_hbm)


SEC = 16
NSEC = NCH // SEC
TAIL = NCH - NSEC * SEC


def _sc_pass_b(g1_hbm, g2_hbm, send3d_hbm, rec3d_hbm):
    mesh = plsc.VectorSubcoreMesh(core_axis_name="c", subcore_axis_name="s")

    @functools.partial(
        pl.kernel,
        mesh=mesh,
        out_type=jax.ShapeDtypeStruct((NC * N, D), jnp.float32),
        scratch_types=[
            pltpu.VMEM_SHARED((N, D), jnp.float32),
            pltpu.VMEM((SEC, CH), jnp.int32),
            pltpu.VMEM((NCH, CH), jnp.int32),
            pltpu.VMEM((CH, D), jnp.float32),
            pltpu.VMEM((CH, D), jnp.float32),
            pltpu.SemaphoreType.DMA,
            pltpu.SemaphoreType.DMA,
        ],
    )
    def k(g1, g2, send3d, rec3d, out, acc, sidx, ridx, rowsA, rowsB, semA, semB):
        cid = lax.axis_index("c")
        sid = lax.axis_index("s")
        wid = cid * NS + sid

        _zero_fill(rowsA, CH, D)

        def zinit(j, _):
            rc = sid + j * NS

            @pl.when(rc < NRC)
            def _():
                pltpu.sync_copy(rowsA, acc.at[pl.ds(rc * RCH, RCH)])

            return 0

        lax.fori_loop(0, RITER, zinit, 0)
        plsc.subcore_barrier()

        pltpu.sync_copy(rec3d.at[wid], ridx)

        def run_section(t, nch_sec):
            c0 = t * SEC
            bufs = [(rowsA, semA), (rowsB, semB)]
            cp = pltpu.async_copy(g1.at[sidx.at[0]], rowsA, semA)
            for i in range(nch_sec):
                buf, _ = bufs[i % 2]
                nbuf, nsem = bufs[(i + 1) % 2]
                ncp = None
                if i + 1 < nch_sec:
                    ncp = pltpu.async_copy(g1.at[sidx.at[i + 1]], nbuf, nsem)
                cp.wait()
                pltpu.sync_copy(buf, acc.at[ridx.at[c0 + i]], add=True)
                cp = ncp

        def sect(t, _):
            pltpu.sync_copy(send3d.at[wid, pl.ds(t * SEC, SEC)], sidx)
            run_section(t, SEC)
            return 0

        lax.fori_loop(0, NSEC, sect, 0)
        pltpu.sync_copy(
            send3d.at[wid, pl.ds(NSEC * SEC, TAIL)], sidx.at[pl.ds(0, TAIL)]
        )
        run_section(NSEC, TAIL)

        pltpu.async_copy(g2.at[ridx.at[0]], rowsA, semA)

        def pair(j, _):
            c0 = 2 * j
            c1 = c0 + 1
            pltpu.async_copy(g2.at[ridx.at[c1]], rowsB, semB)
            pltpu.make_async_copy(g2.at[ridx.at[0]], rowsA, semA).wait()
            pltpu.sync_copy(rowsA, acc.at[ridx.at[c0]], add=True)
            pltpu.async_copy(g2.at[ridx.at[c0 + 2]], rowsA, semA)
            pltpu.make_async_copy(g2.at[ridx.at[0]], rowsB, semB).wait()
            pltpu.sync_copy(rowsB, acc.at[ridx.at[c1]], add=True)
            return 0

        lax.fori_loop(0, NCH // 2, pair, 0)
        pltpu.make_async_copy(g2.at[ridx.at[0]], rowsA, semA).wait()
        pltpu.sync_copy(rowsA, acc.at[ridx.at[NCH - 1]], add=True)
        plsc.subcore_barrier()

        def wout(j, _):
            rc = sid + j * NS

            @pl.when(rc < NRC)
            def _():
                pltpu.sync_copy(acc.at[pl.ds(rc * RCH, RCH)], rowsA)
                pltpu.sync_copy(rowsA, out.at[pl.ds(cid * N + rc * RCH, RCH)])

            return 0

        lax.fori_loop(0, RITER, wout, 0)

    return k(g1_hbm, g2_hbm, send3d_hbm, rec3d_hbm)


def _tc_prep(h, W_msg, b_msg2, W_upd):
    def body(h_ref, wm_ref, bm_ref, wu_ref, g1_ref, g2_ref):
        Wu_m = wu_ref[D : 2 * D, :]
        Wsp = jnp.dot(wm_ref[0:D, :], Wu_m, preferred_element_type=jnp.float32)
        Wrp = jnp.dot(wm_ref[D : 2 * D, :], Wu_m, preferred_element_type=jnp.float32)
        bp = jnp.dot(bm_ref[...], Wu_m, preferred_element_type=jnp.float32)
        hv = h_ref[...]
        g1_ref[...] = jnp.dot(hv, Wsp, preferred_element_type=jnp.float32)
        g2_ref[...] = jnp.dot(hv, Wrp, preferred_element_type=jnp.float32) + bp

    return pl.pallas_call(
        body,
        out_shape=(
            jax.ShapeDtypeStruct((N, D), jnp.float32),
            jax.ShapeDtypeStruct((N, D), jnp.float32),
        ),
    )(h, W_msg, b_msg2, W_upd)


def _tc_final(h, pA, pB, W_msg, W_upd, b_upd2):
    def body(h_ref, pA_ref, pB_ref, wm_ref, wu_ref, bu_ref, out_ref):
        Wu_h = wu_ref[0:D, :]
        Wu_m = wu_ref[D : 2 * D, :]
        Wep = jnp.dot(wm_ref[2 * D : 3 * D, :], Wu_m, preferred_element_type=jnp.float32)
        S3 = pA_ref[0:N, :] + pA_ref[N : 2 * N, :]
        Sg12 = pB_ref[0:N, :] + pB_ref[N : 2 * N, :]
        out = (
            jnp.dot(h_ref[...], Wu_h, preferred_element_type=jnp.float32)
            + Sg12
            + jnp.dot(S3, Wep, preferred_element_type=jnp.float32)
            + bu_ref[...]
        )
        out_ref[...] = out

    return pl.pallas_call(
        body,
        out_shape=jax.ShapeDtypeStruct((N, D), jnp.float32),
    )(h, pA, pB, W_msg, W_upd, b_upd2)


def kernel(h, edge_index, edge_attr, W_msg, b_msg, W_upd, b_upd):
    send = edge_index[0].astype(jnp.int32)
    rec = edge_index[1].astype(jnp.int32)
    send3d = send.reshape(NW, NCH, CH)
    rec3d = rec.reshape(NW, NCH, CH)

    g1, g2 = _tc_prep(h, W_msg, b_msg.reshape(1, D), W_upd)
    pA = _sc_pass_a(edge_attr, rec3d)
    pB = _sc_pass_b(g1, g2, send3d, rec3d)
    return _tc_final(h, pA, pB, W_msg, W_upd, b_upd.reshape(1, D))


# speedup vs baseline: 8.4800x; 1.1746x over previous
import functools

import jax
import jax.numpy as jnp
from jax import lax
from jax.experimental import pallas as pl
from jax.experimental.pallas import tpu as pltpu
from jax.experimental.pallas import tpu_sc as plsc

N = 10000
E = 320000
D = 128

NC = 2
NS = 16
NW = NC * NS
EPW = E // NW
CH = 80
NCH = EPW // CH
RCH = 80
NRC = N // RCH
RITER = -(-NRC // NS)


def _zero_fill(zbuf, rows, width):
    lanes = width // 16

    def body(i, _):
        r = i // lanes
        c = (i % lanes) * 16
        zbuf[r, pl.ds(c, 16)] = jnp.zeros((16,), jnp.float32)
        return 0

    lax.fori_loop(0, rows * lanes, body, 0)


def _sc_pass_a(ea_hbm, rec3d_hbm):
    mesh = plsc.VectorSubcoreMesh(core_axis_name="c", subcore_axis_name="s")

    @functools.partial(
        pl.kernel,
        mesh=mesh,
        out_type=jax.ShapeDtypeStruct((NC * N, D), jnp.float32),
        scratch_types=[
            pltpu.VMEM_SHARED((N, D), jnp.float32),
            pltpu.VMEM((NCH, CH), jnp.int32),
            pltpu.VMEM((CH, D), jnp.float32),
            pltpu.VMEM((CH, D), jnp.float32),
            pltpu.VMEM((CH, D), jnp.float32),
            pltpu.SemaphoreType.DMA,
            pltpu.SemaphoreType.DMA,
            pltpu.SemaphoreType.DMA,
        ],
    )
    def k(ea, rec3d, out, acc, ridx, b0, b1, b2, s0, s1, s2):
        cid = lax.axis_index("c")
        sid = lax.axis_index("s")
        wid = cid * NS + sid
        base = wid * EPW
        bufs = [(b0, s0), (b1, s1), (b2, s2)]

        _zero_fill(b0, CH, D)

        def zinit(j, _):
            rc = sid + j * NS

            @pl.when(rc < NRC)
            def _():
                pltpu.sync_copy(b0, acc.at[pl.ds(rc * RCH, RCH)])

            return 0

        lax.fori_loop(0, RITER, zinit, 0)
        plsc.subcore_barrier()

        pltpu.sync_copy(rec3d.at[wid], ridx)

        def issue(c, slot):
            buf, sem = bufs[slot]
            pltpu.async_copy(ea.at[pl.ds(base + c * CH, CH)], buf, sem)

        def drain(c, slot):
            buf, sem = bufs[slot]
            pltpu.make_async_copy(ea.at[pl.ds(base, CH)], buf, sem).wait()
            pltpu.sync_copy(buf, acc.at[ridx.at[c]], add=True)

        issue(0, 0)
        issue(1, 1)

        def ring(j, _):
            c = 3 * j
            issue(c + 2, 2)
            drain(c, 0)
            issue(c + 3, 0)
            drain(c + 1, 1)
            issue(c + 4, 1)
            drain(c + 2, 2)
            return 0

        lax.fori_loop(0, NCH // 3, ring, 0)
        drain(NCH - 2, (NCH - 2) % 3)
        drain(NCH - 1, (NCH - 1) % 3)
        plsc.subcore_barrier()

        def wout(j, _):
            rc = sid + j * NS

            @pl.when(rc < NRC)
            def _():
                pltpu.sync_copy(acc.at[pl.ds(rc * RCH, RCH)], b0)
                pltpu.sync_copy(b0, out.at[pl.ds(cid * N + rc * RCH, RCH)])

            return 0

        lax.fori_loop(0, RITER, wout, 0)

    return k(ea_hbm, rec3d_hbm)


SEC = 16
NSEC = NCH // SEC
TAIL = NCH - NSEC * SEC


def _sc_pass_b(g1_hbm, g2_hbm, send3d_hbm, rec3d_hbm):
    mesh = plsc.VectorSubcoreMesh(core_axis_name="c", subcore_axis_name="s")

    @functools.partial(
        pl.kernel,
        mesh=mesh,
        out_type=jax.ShapeDtypeStruct((NC * N, D), jnp.float32),
        scratch_types=[
            pltpu.VMEM_SHARED((N, D), jnp.float32),
            pltpu.VMEM((SEC, CH), jnp.int32),
            pltpu.VMEM((NCH, CH), jnp.int32),
            pltpu.VMEM((CH, D), jnp.float32),
            pltpu.VMEM((CH, D), jnp.float32),
            pltpu.VMEM((CH, D), jnp.float32),
            pltpu.SemaphoreType.DMA,
            pltpu.SemaphoreType.DMA,
            pltpu.SemaphoreType.DMA,
        ],
    )
    def k(g1, g2, send3d, rec3d, out, acc, sidx, ridx, b0, b1, b2, s0, s1, s2):
        cid = lax.axis_index("c")
        sid = lax.axis_index("s")
        wid = cid * NS + sid
        bufs = [(b0, s0), (b1, s1), (b2, s2)]

        _zero_fill(b0, CH, D)

        def zinit(j, _):
            rc = sid + j * NS

            @pl.when(rc < NRC)
            def _():
                pltpu.sync_copy(b0, acc.at[pl.ds(rc * RCH, RCH)])

            return 0

        lax.fori_loop(0, RITER, zinit, 0)
        plsc.subcore_barrier()

        pltpu.sync_copy(rec3d.at[wid], ridx)

        def run_section(c0, n):
            def issue(i):
                buf, sem = bufs[i % 3]
                pltpu.async_copy(g1.at[sidx.at[i]], buf, sem)

            issue(0)
            if n > 1:
                issue(1)
            for i in range(n):
                buf, sem = bufs[i % 3]
                if i + 2 < n:
                    issue(i + 2)
                pltpu.make_async_copy(g1.at[sidx.at[0]], buf, sem).wait()
                pltpu.sync_copy(buf, acc.at[ridx.at[c0 + i]], add=True)

        def sect(t, _):
            pltpu.sync_copy(send3d.at[wid, pl.ds(t * SEC, SEC)], sidx)
            run_section(t * SEC, SEC)
            return 0

        lax.fori_loop(0, NSEC, sect, 0)
        pltpu.sync_copy(
            send3d.at[wid, pl.ds(NSEC * SEC, TAIL)], sidx.at[pl.ds(0, TAIL)]
        )
        run_section(NSEC * SEC, TAIL)

        def issue2(c, slot):
            buf, sem = bufs[slot]
            pltpu.async_copy(g2.at[ridx.at[c]], buf, sem)

        def drain2(c, slot):
            buf, sem = bufs[slot]
            pltpu.make_async_copy(g2.at[ridx.at[0]], buf, sem).wait()
            pltpu.sync_copy(buf, acc.at[ridx.at[c]], add=True)

        issue2(0, 0)
        issue2(1, 1)

        def ring(j, _):
            c = 3 * j
            issue2(c + 2, 2)
            drain2(c, 0)
            issue2(c + 3, 0)
            drain2(c + 1, 1)
            issue2(c + 4, 1)
            drain2(c + 2, 2)
            return 0

        lax.fori_loop(0, NCH // 3, ring, 0)
        drain2(NCH - 2, (NCH - 2) % 3)
        drain2(NCH - 1, (NCH - 1) % 3)
        plsc.subcore_barrier()

        def wout(j, _):
            rc = sid + j * NS

            @pl.when(rc < NRC)
            def _():
                pltpu.sync_copy(acc.at[pl.ds(rc * RCH, RCH)], b0)
                pltpu.sync_copy(b0, out.at[pl.ds(cid * N + rc * RCH, RCH)])

            return 0

        lax.fori_loop(0, RITER, wout, 0)

    return k(g1_hbm, g2_hbm, send3d_hbm, rec3d_hbm)


def _tc_prep(h, W_msg, b_msg2, W_upd):
    def body(h_ref, wm_ref, bm_ref, wu_ref, g1_ref, g2_ref):
        Wu_m = wu_ref[D : 2 * D, :]
        Wsp = jnp.dot(wm_ref[0:D, :], Wu_m, preferred_element_type=jnp.float32)
        Wrp = jnp.dot(wm_ref[D : 2 * D, :], Wu_m, preferred_element_type=jnp.float32)
        bp = jnp.dot(bm_ref[...], Wu_m, preferred_element_type=jnp.float32)
        hv = h_ref[...]
        g1_ref[...] = jnp.dot(hv, Wsp, preferred_element_type=jnp.float32)
        g2_ref[...] = jnp.dot(hv, Wrp, preferred_element_type=jnp.float32) + bp

    return pl.pallas_call(
        body,
        out_shape=(
            jax.ShapeDtypeStruct((N, D), jnp.float32),
            jax.ShapeDtypeStruct((N, D), jnp.float32),
        ),
    )(h, W_msg, b_msg2, W_upd)


def _tc_final(h, pA, pB, W_msg, W_upd, b_upd2):
    def body(h_ref, pA_ref, pB_ref, wm_ref, wu_ref, bu_ref, out_ref):
        Wu_h = wu_ref[0:D, :]
        Wu_m = wu_ref[D : 2 * D, :]
        Wep = jnp.dot(wm_ref[2 * D : 3 * D, :], Wu_m, preferred_element_type=jnp.float32)
        S3 = pA_ref[0:N, :] + pA_ref[N : 2 * N, :]
        Sg12 = pB_ref[0:N, :] + pB_ref[N : 2 * N, :]
        out = (
            jnp.dot(h_ref[...], Wu_h, preferred_element_type=jnp.float32)
            + Sg12
            + jnp.dot(S3, Wep, preferred_element_type=jnp.float32)
            + bu_ref[...]
        )
        out_ref[...] = out

    return pl.pallas_call(
        body,
        out_shape=jax.ShapeDtypeStruct((N, D), jnp.float32),
    )(h, pA, pB, W_msg, W_upd, b_upd2)


def kernel(h, edge_index, edge_attr, W_msg, b_msg, W_upd, b_upd):
    send = edge_index[0].astype(jnp.int32)
    rec = edge_index[1].astype(jnp.int32)
    send3d = send.reshape(NW, NCH, CH)
    rec3d = rec.reshape(NW, NCH, CH)

    g1, g2 = _tc_prep(h, W_msg, b_msg.reshape(1, D), W_upd)
    pA = _sc_pass_a(edge_attr, rec3d)
    pB = _sc_pass_b(g1, g2, send3d, rec3d)
    return _tc_final(h, pA, pB, W_msg, W_upd, b_upd.reshape(1, D))


# speedup vs baseline: 9.0519x; 1.0674x over previous
import functools

import jax
import jax.numpy as jnp
from jax import lax
from jax.experimental import pallas as pl
from jax.experimental.pallas import tpu as pltpu
from jax.experimental.pallas import tpu_sc as plsc

N = 10000
E = 320000
D = 128

NC = 2
NS = 16
NW = NC * NS
EPW = E // NW
CH = 80
NCH = EPW // CH
RCH = 80
NRC = N // RCH
RITER = -(-NRC // NS)
SEC = 24
NSEC = NCH // SEC
TAIL = NCH - NSEC * SEC


def _zero_fill(zbuf, rows, width):
    lanes = width // 16

    def body(i, _):
        r = i // lanes
        c = (i % lanes) * 16
        zbuf[r, pl.ds(c, 16)] = jnp.zeros((16,), jnp.float32)
        return 0

    lax.fori_loop(0, rows * lanes, body, 0)


def _sc_passes(ea_hbm, g1_hbm, g2_hbm, ei4d_hbm):
    mesh = plsc.VectorSubcoreMesh(core_axis_name="c", subcore_axis_name="s")

    @functools.partial(
        pl.kernel,
        mesh=mesh,
        out_type=(
            jax.ShapeDtypeStruct((NC * N, D), jnp.float32),
            jax.ShapeDtypeStruct((NC * N, D), jnp.float32),
        ),
        scratch_types=[
            pltpu.VMEM_SHARED((N, D), jnp.float32),
            pltpu.VMEM((SEC, CH), jnp.int32),
            pltpu.VMEM((NCH, CH), jnp.int32),
            pltpu.VMEM((CH, D), jnp.float32),
            pltpu.VMEM((CH, D), jnp.float32),
            pltpu.VMEM((CH, D), jnp.float32),
            pltpu.SemaphoreType.DMA,
            pltpu.SemaphoreType.DMA,
            pltpu.SemaphoreType.DMA,
        ],
    )
    def k(ea, g1, g2, ei4d, outA, outB, acc, sidx, ridx, b0, b1, b2, s0, s1, s2):
        cid = lax.axis_index("c")
        sid = lax.axis_index("s")
        wid = cid * NS + sid
        base = wid * EPW
        bufs = [(b0, s0), (b1, s1), (b2, s2)]

        _zero_fill(b0, CH, D)

        def zinit(j, _):
            rc = sid + j * NS

            @pl.when(rc < NRC)
            def _():
                pltpu.sync_copy(b0, acc.at[pl.ds(rc * RCH, RCH)])

            return 0

        def wout(out):
            def w(j, _):
                rc = sid + j * NS

                @pl.when(rc < NRC)
                def _():
                    pltpu.sync_copy(acc.at[pl.ds(rc * RCH, RCH)], b0)
                    pltpu.sync_copy(b0, out.at[pl.ds(cid * N + rc * RCH, RCH)])

                return 0

            lax.fori_loop(0, RITER, w, 0)

        lax.fori_loop(0, RITER, zinit, 0)
        plsc.subcore_barrier()

        pltpu.sync_copy(ei4d.at[1, wid], ridx)

        def issue_a(c, slot):
            buf, sem = bufs[slot]
            pltpu.async_copy(ea.at[pl.ds(base + c * CH, CH)], buf, sem)

        def drain_a(c, slot):
            buf, sem = bufs[slot]
            pltpu.make_async_copy(ea.at[pl.ds(base, CH)], buf, sem).wait()
            pltpu.sync_copy(buf, acc.at[ridx.at[c]], add=True)

        issue_a(0, 0)
        issue_a(1, 1)

        def ring_a(j, _):
            c = 3 * j
            issue_a(c + 2, 2)
            drain_a(c, 0)
            issue_a(c + 3, 0)
            drain_a(c + 1, 1)
            issue_a(c + 4, 1)
            drain_a(c + 2, 2)
            return 0

        lax.fori_loop(0, NCH // 3, ring_a, 0)
        drain_a(NCH - 2, (NCH - 2) % 3)
        drain_a(NCH - 1, (NCH - 1) % 3)
        plsc.subcore_barrier()

        _zero_fill(b1, CH, D)

        def wz(j, _):
            rc = sid + j * NS

            @pl.when(rc < NRC)
            def _():
                pltpu.sync_copy(acc.at[pl.ds(rc * RCH, RCH)], b0)
                pltpu.sync_copy(b0, outA.at[pl.ds(cid * N + rc * RCH, RCH)])
                pltpu.sync_copy(b1, acc.at[pl.ds(rc * RCH, RCH)])

            return 0

        lax.fori_loop(0, RITER, wz, 0)
        plsc.subcore_barrier()

        def run_section(c0, n):
            def issue(i):
                buf, sem = bufs[i % 3]
                pltpu.async_copy(g1.at[sidx.at[i]], buf, sem)

            issue(0)
            if n > 1:
                issue(1)
            for i in range(n):
                buf, sem = bufs[i % 3]
                if i + 2 < n:
                    issue(i + 2)
                pltpu.make_async_copy(g1.at[sidx.at[0]], buf, sem).wait()
                pltpu.sync_copy(buf, acc.at[ridx.at[c0 + i]], add=True)

        def sect(t, _):
            pltpu.sync_copy(ei4d.at[0, wid, pl.ds(t * SEC, SEC)], sidx)
            run_section(t * SEC, SEC)
            return 0

        lax.fori_loop(0, NSEC, sect, 0)
        pltpu.sync_copy(
            ei4d.at[0, wid, pl.ds(NSEC * SEC, TAIL)], sidx.at[pl.ds(0, TAIL)]
        )
        run_section(NSEC * SEC, TAIL)

        def issue_b(c, slot):
            buf, sem = bufs[slot]
            pltpu.async_copy(g2.at[ridx.at[c]], buf, sem)

        def drain_b(c, slot):
            buf, sem = bufs[slot]
            pltpu.make_async_copy(g2.at[ridx.at[0]], buf, sem).wait()
            pltpu.sync_copy(buf, acc.at[ridx.at[c]], add=True)

        issue_b(0, 0)
        issue_b(1, 1)

        def ring_b(j, _):
            c = 3 * j
            issue_b(c + 2, 2)
            drain_b(c, 0)
            issue_b(c + 3, 0)
            drain_b(c + 1, 1)
            issue_b(c + 4, 1)
            drain_b(c + 2, 2)
            return 0

        lax.fori_loop(0, NCH // 3, ring_b, 0)
        drain_b(NCH - 2, (NCH - 2) % 3)
        drain_b(NCH - 1, (NCH - 1) % 3)
        plsc.subcore_barrier()

        wout(outB)

    return k(ea_hbm, g1_hbm, g2_hbm, ei4d_hbm)


def _tc_prep(h, W_msg, b_msg2, W_upd):
    def body(h_ref, wm_ref, bm_ref, wu_ref, g1_ref, g2_ref):
        Wu_m = wu_ref[D : 2 * D, :]
        Wsp = jnp.dot(wm_ref[0:D, :], Wu_m, preferred_element_type=jnp.float32)
        Wrp = jnp.dot(wm_ref[D : 2 * D, :], Wu_m, preferred_element_type=jnp.float32)
        bp = jnp.dot(bm_ref[...], Wu_m, preferred_element_type=jnp.float32)
        hv = h_ref[...]
        g1_ref[...] = jnp.dot(hv, Wsp, preferred_element_type=jnp.float32)
        g2_ref[...] = jnp.dot(hv, Wrp, preferred_element_type=jnp.float32) + bp

    return pl.pallas_call(
        body,
        out_shape=(
            jax.ShapeDtypeStruct((N, D), jnp.float32),
            jax.ShapeDtypeStruct((N, D), jnp.float32),
        ),
    )(h, W_msg, b_msg2, W_upd)


def _tc_final(h, pA, pB, W_msg, W_upd, b_upd2):
    def body(h_ref, pA_ref, pB_ref, wm_ref, wu_ref, bu_ref, out_ref):
        Wu_h = wu_ref[0:D, :]
        Wu_m = wu_ref[D : 2 * D, :]
        Wep = jnp.dot(wm_ref[2 * D : 3 * D, :], Wu_m, preferred_element_type=jnp.float32)
        S3 = pA_ref[0:N, :] + pA_ref[N : 2 * N, :]
        Sg12 = pB_ref[0:N, :] + pB_ref[N : 2 * N, :]
        out = (
            jnp.dot(h_ref[...], Wu_h, preferred_element_type=jnp.float32)
            + Sg12
            + jnp.dot(S3, Wep, preferred_element_type=jnp.float32)
            + bu_ref[...]
        )
        out_ref[...] = out

    return pl.pallas_call(
        body,
        out_shape=jax.ShapeDtypeStruct((N, D), jnp.float32),
    )(h, pA, pB, W_msg, W_upd, b_upd2)


def kernel(h, edge_index, edge_attr, W_msg, b_msg, W_upd, b_upd):
    ei4d = edge_index.astype(jnp.int32).reshape(2, NW, NCH, CH)

    g1, g2 = _tc_prep(h, W_msg, b_msg.reshape(1, D), W_upd)
    pA, pB = _sc_passes(edge_attr, g1, g2, ei4d)
    return _tc_final(h, pA, pB, W_msg, W_upd, b_upd.reshape(1, D))


# speedup vs baseline: 9.1030x; 1.0056x over previous
import functools

import jax
import jax.numpy as jnp
from jax import lax
from jax.experimental import pallas as pl
from jax.experimental.pallas import tpu as pltpu
from jax.experimental.pallas import tpu_sc as plsc

N = 10000
E = 320000
D = 128

NC = 2
NS = 16
NW = NC * NS
EPW = E // NW
CH = 80
NCH = EPW // CH
RCH = 80
NRC = N // RCH
RITER = -(-NRC // NS)
SEC = 24
NSEC = NCH // SEC
TAIL = NCH - NSEC * SEC


def _zero_fill(zbuf, rows, width):
    lanes = width // 16

    def body(i, _):
        r = i // lanes
        c = (i % lanes) * 16
        zbuf[r, pl.ds(c, 16)] = jnp.zeros((16,), jnp.float32)
        return 0

    lax.fori_loop(0, rows * lanes, body, 0)


def _sc_passes(ea_hbm, g1_hbm, g2_hbm, ei4d_hbm):
    mesh = plsc.VectorSubcoreMesh(core_axis_name="c", subcore_axis_name="s")

    @functools.partial(
        pl.kernel,
        mesh=mesh,
        out_type=(
            jax.ShapeDtypeStruct((NC * N, D), jnp.float32),
            jax.ShapeDtypeStruct((NC * N, D), jnp.float32),
        ),
        scratch_types=[
            pltpu.VMEM_SHARED((N, D), jnp.float32),
            pltpu.VMEM((SEC, CH), jnp.int32),
            pltpu.VMEM((NCH, CH), jnp.int32),
            pltpu.VMEM((CH, D), jnp.float32),
            pltpu.VMEM((CH, D), jnp.float32),
            pltpu.VMEM((CH, D), jnp.float32),
            pltpu.SemaphoreType.DMA,
            pltpu.SemaphoreType.DMA,
            pltpu.SemaphoreType.DMA,
        ],
    )
    def k(ea, g1, g2, ei4d, outA, outB, acc, sidx, ridx, b0, b1, b2, s0, s1, s2):
        cid = lax.axis_index("c")
        sid = lax.axis_index("s")
        wid = cid * NS + sid
        base = wid * EPW
        bufs = [(b0, s0), (b1, s1), (b2, s2)]

        _zero_fill(b0, CH, D)

        def zinit(j, _):
            rc = sid + j * NS

            @pl.when(rc < NRC)
            def _():
                pltpu.sync_copy(b0, acc.at[pl.ds(rc * RCH, RCH)])

            return 0

        def wout(out):
            def w(j, _):
                rc = sid + j * NS

                @pl.when(rc < NRC)
                def _():
                    pltpu.sync_copy(
                        acc.at[pl.ds(rc * RCH, RCH)],
                        out.at[pl.ds(cid * N + rc * RCH, RCH)],
                    )

                return 0

            lax.fori_loop(0, RITER, w, 0)

        lax.fori_loop(0, RITER, zinit, 0)
        plsc.subcore_barrier()

        pltpu.sync_copy(ei4d.at[1, wid], ridx)

        def issue_a(c, slot):
            buf, sem = bufs[slot]
            pltpu.async_copy(ea.at[pl.ds(base + c * CH, CH)], buf, sem)

        def drain_a(c, slot):
            buf, sem = bufs[slot]
            pltpu.make_async_copy(ea.at[pl.ds(base, CH)], buf, sem).wait()
            pltpu.sync_copy(buf, acc.at[ridx.at[c]], add=True)

        issue_a(0, 0)
        issue_a(1, 1)

        def ring_a(j, _):
            c = 3 * j
            issue_a(c + 2, 2)
            drain_a(c, 0)
            issue_a(c + 3, 0)
            drain_a(c + 1, 1)
            issue_a(c + 4, 1)
            drain_a(c + 2, 2)
            return 0

        lax.fori_loop(0, NCH // 3, ring_a, 0)
        drain_a(NCH - 2, (NCH - 2) % 3)
        drain_a(NCH - 1, (NCH - 1) % 3)
        plsc.subcore_barrier()

        _zero_fill(b1, CH, D)

        def wz(j, _):
            rc = sid + j * NS

            @pl.when(rc < NRC)
            def _():
                pltpu.sync_copy(
                    acc.at[pl.ds(rc * RCH, RCH)],
                    outA.at[pl.ds(cid * N + rc * RCH, RCH)],
                )
                pltpu.sync_copy(b1, acc.at[pl.ds(rc * RCH, RCH)])

            return 0

        lax.fori_loop(0, RITER, wz, 0)
        plsc.subcore_barrier()

        def run_section(c0, n):
            def issue(i):
                buf, sem = bufs[i % 3]
                pltpu.async_copy(g1.at[sidx.at[i]], buf, sem)

            issue(0)
            if n > 1:
                issue(1)
            for i in range(n):
                buf, sem = bufs[i % 3]
                if i + 2 < n:
                    issue(i + 2)
                pltpu.make_async_copy(g1.at[sidx.at[0]], buf, sem).wait()
                pltpu.sync_copy(buf, acc.at[ridx.at[c0 + i]], add=True)

        def sect(t, _):
            pltpu.sync_copy(ei4d.at[0, wid, pl.ds(t * SEC, SEC)], sidx)
            run_section(t * SEC, SEC)
            return 0

        lax.fori_loop(0, NSEC, sect, 0)
        pltpu.sync_copy(
            ei4d.at[0, wid, pl.ds(NSEC * SEC, TAIL)], sidx.at[pl.ds(0, TAIL)]
        )
        run_section(NSEC * SEC, TAIL)

        def issue_b(c, slot):
            buf, sem = bufs[slot]
            pltpu.async_copy(g2.at[ridx.at[c]], buf, sem)

        def drain_b(c, slot):
            buf, sem = bufs[slot]
            pltpu.make_async_copy(g2.at[ridx.at[0]], buf, sem).wait()
            pltpu.sync_copy(buf, acc.at[ridx.at[c]], add=True)

        issue_b(0, 0)
        issue_b(1, 1)

        def ring_b(j, _):
            c = 3 * j
            issue_b(c + 2, 2)
            drain_b(c, 0)
            issue_b(c + 3, 0)
            drain_b(c + 1, 1)
            issue_b(c + 4, 1)
            drain_b(c + 2, 2)
            return 0

        lax.fori_loop(0, NCH // 3, ring_b, 0)
        drain_b(NCH - 2, (NCH - 2) % 3)
        drain_b(NCH - 1, (NCH - 1) % 3)
        plsc.subcore_barrier()

        wout(outB)

    return k(ea_hbm, g1_hbm, g2_hbm, ei4d_hbm)


def _tc_prep(h, W_msg, b_msg2, W_upd):
    def body(h_ref, wm_ref, bm_ref, wu_ref, g1_ref, g2_ref):
        Wu_m = wu_ref[D : 2 * D, :]
        Wsp = jnp.dot(wm_ref[0:D, :], Wu_m, preferred_element_type=jnp.float32)
        Wrp = jnp.dot(wm_ref[D : 2 * D, :], Wu_m, preferred_element_type=jnp.float32)
        bp = jnp.dot(bm_ref[...], Wu_m, preferred_element_type=jnp.float32)
        hv = h_ref[...]
        g1_ref[...] = jnp.dot(hv, Wsp, preferred_element_type=jnp.float32)
        g2_ref[...] = jnp.dot(hv, Wrp, preferred_element_type=jnp.float32) + bp

    return pl.pallas_call(
        body,
        out_shape=(
            jax.ShapeDtypeStruct((N, D), jnp.float32),
            jax.ShapeDtypeStruct((N, D), jnp.float32),
        ),
    )(h, W_msg, b_msg2, W_upd)


def _tc_final(h, pA, pB, W_msg, W_upd, b_upd2):
    def body(h_ref, pA_ref, pB_ref, wm_ref, wu_ref, bu_ref, out_ref):
        Wu_h = wu_ref[0:D, :]
        Wu_m = wu_ref[D : 2 * D, :]
        Wep = jnp.dot(wm_ref[2 * D : 3 * D, :], Wu_m, preferred_element_type=jnp.float32)
        S3 = pA_ref[0:N, :] + pA_ref[N : 2 * N, :]
        Sg12 = pB_ref[0:N, :] + pB_ref[N : 2 * N, :]
        out = (
            jnp.dot(h_ref[...], Wu_h, preferred_element_type=jnp.float32)
            + Sg12
            + jnp.dot(S3, Wep, preferred_element_type=jnp.float32)
            + bu_ref[...]
        )
        out_ref[...] = out

    return pl.pallas_call(
        body,
        out_shape=jax.ShapeDtypeStruct((N, D), jnp.float32),
    )(h, pA, pB, W_msg, W_upd, b_upd2)


def kernel(h, edge_index, edge_attr, W_msg, b_msg, W_upd, b_upd):
    ei4d = edge_index.astype(jnp.int32).reshape(2, NW, NCH, CH)

    g1, g2 = _tc_prep(h, W_msg, b_msg.reshape(1, D), W_upd)
    pA, pB = _sc_passes(edge_attr, g1, g2, ei4d)
    return _tc_final(h, pA, pB, W_msg, W_upd, b_upd.reshape(1, D))
